# baseline (speedup 1.0000x reference)
import functools

import jax
import jax.numpy as jnp
from jax.experimental import pallas as pl
from jax.experimental.pallas import tpu as pltpu


def _fused_kernel(x_hbm, w_ref, b_ref, o_hbm,
                  x_buf, o_buf, g_buf, in_sem, out_sem,
                  *, iblock: int, oblock: int, n_osteps: int,
                  in_slots: int, out_slots: int):
    sub = iblock // oblock

    def dma_in(slot, istep):
        pltpu.make_async_copy(
            x_hbm.at[pl.ds(istep * iblock, iblock), :],
            x_buf.at[slot], in_sem.at[slot]).start()

    def wait_in(slot):
        pltpu.make_async_copy(
            x_hbm.at[pl.ds(0, iblock), :],
            x_buf.at[slot], in_sem.at[slot]).wait()

    def dma_out(slot, ostep):
        pltpu.make_async_copy(
            o_buf.at[slot],
            o_hbm.at[pl.ds(ostep * oblock, oblock), :], out_sem.at[slot]).start()

    def wait_out(slot):
        pltpu.make_async_copy(
            o_buf.at[slot],
            o_hbm.at[pl.ds(0, oblock), :], out_sem.at[slot]).wait()

    n_isteps = n_osteps // sub
    for s in range(min(in_slots - 1, n_isteps)):
        dma_in(s, s)
    w = w_ref[...].astype(jnp.bfloat16)
    g_buf[...] = jax.lax.dot_general(
        w, w, (((1,), (1,)), ((), ())),
        preferred_element_type=jnp.float32).astype(jnp.bfloat16)

    def body(ostep, _):
        istep = ostep // sub
        part = jax.lax.rem(ostep, sub)
        cur_in = jax.lax.rem(istep, in_slots)
        cur_out = jax.lax.rem(ostep, out_slots)

        @pl.when((part == 0) & (istep + in_slots - 1 < n_isteps))
        def _():
            dma_in(jax.lax.rem(istep + in_slots - 1, in_slots),
                   istep + in_slots - 1)

        @pl.when(part == 0)
        def _():
            wait_in(cur_in)

        @pl.when(ostep >= out_slots)
        def _():
            wait_out(cur_out)

        xt = x_buf[cur_in, pl.ds(part * oblock, oblock), :].astype(jnp.bfloat16)
        y = jnp.dot(xt, g_buf[...], preferred_element_type=jnp.float32)
        o_buf[cur_out] = jnp.maximum(y + b_ref[...], 0.0).astype(o_buf.dtype)

        dma_out(cur_out, ostep)
        return ()

    jax.lax.fori_loop(0, n_osteps, body, (), unroll=False)
    for s in range(max(n_osteps - out_slots, 0), n_osteps):
        wait_out(s % out_slots)


def kernel(x, w1, b):
    n_data, n_feat = x.shape
    nf_w, n_hidden = w1.shape
    assert nf_w == n_feat

    b2d = b.reshape(1, n_feat)

    iblock = n_data
    for cand in (1024, 512, 256, 128, 64, 32, 16, 8):
        if n_data % cand == 0:
            iblock = cand
            break
    oblock = max(iblock // 2, 8)
    n_osteps = n_data // oblock
    in_slots = min(5, max(n_data // iblock, 2))
    out_slots = 6

    cost = pl.CostEstimate(
        flops=2 * n_data * n_feat * n_feat + 2 * n_feat * n_feat * n_hidden,
        transcendentals=0,
        bytes_accessed=2 * n_data * n_feat * 4 + n_feat * n_hidden * 4,
    )
    return pl.pallas_call(
        functools.partial(_fused_kernel, iblock=iblock, oblock=oblock,
                          n_osteps=n_osteps, in_slots=in_slots,
                          out_slots=out_slots),
        out_shape=jax.ShapeDtypeStruct((n_data, n_feat), x.dtype),
        in_specs=[
            pl.BlockSpec(memory_space=pltpu.MemorySpace.HBM),
            pl.BlockSpec(memory_space=pltpu.MemorySpace.VMEM),
            pl.BlockSpec(memory_space=pltpu.MemorySpace.VMEM),
        ],
        out_specs=pl.BlockSpec(memory_space=pltpu.MemorySpace.HBM),
        scratch_shapes=[
            pltpu.VMEM((in_slots, iblock, n_feat), x.dtype),
            pltpu.VMEM((out_slots, oblock, n_feat), x.dtype),
            pltpu.VMEM((n_feat, n_feat), jnp.bfloat16),
            pltpu.SemaphoreType.DMA((in_slots,)),
            pltpu.SemaphoreType.DMA((out_slots,)),
        ],
        cost_estimate=cost,
        compiler_params=pltpu.CompilerParams(
            vmem_limit_bytes=58 * 1024 * 1024,
        ),
    )(x, w1, b2d)


# speedup vs baseline: 1.0687x; 1.0687x over previous
import functools

import jax
import jax.numpy as jnp
from jax.experimental import pallas as pl
from jax.experimental.pallas import tpu as pltpu


def _fused_kernel(x_hbm, w_ref, b_ref, o_hbm,
                  x_buf, o_buf, g_buf, in_sem, out_sem,
                  *, block: int, n_steps: int, in_slots: int, out_slots: int):
    def dma_in(slot, step):
        pltpu.make_async_copy(
            x_hbm.at[pl.ds(step * block, block), :],
            x_buf.at[slot], in_sem.at[slot]).start()

    def wait_in(slot):
        pltpu.make_async_copy(
            x_hbm.at[pl.ds(0, block), :],
            x_buf.at[slot], in_sem.at[slot]).wait()

    def dma_out(slot, step):
        pltpu.make_async_copy(
            o_buf.at[slot],
            o_hbm.at[pl.ds(step * block, block), :], out_sem.at[slot]).start()

    def wait_out(slot):
        pltpu.make_async_copy(
            o_buf.at[slot],
            o_hbm.at[pl.ds(0, block), :], out_sem.at[slot]).wait()

    for s in range(min(in_slots - 1, n_steps)):
        dma_in(s, s)
    w = w_ref[...].astype(jnp.bfloat16)
    g_buf[...] = jax.lax.dot_general(
        w, w, (((1,), (1,)), ((), ())),
        preferred_element_type=jnp.float32).astype(jnp.bfloat16)

    def body(step, _):
        cur_in = jax.lax.rem(step, in_slots)
        cur_out = jax.lax.rem(step, out_slots)

        @pl.when(step + in_slots - 1 < n_steps)
        def _():
            dma_in(jax.lax.rem(step + in_slots - 1, in_slots),
                   step + in_slots - 1)

        wait_in(cur_in)

        @pl.when(step >= out_slots)
        def _():
            wait_out(cur_out)

        xt = x_buf[cur_in].astype(jnp.bfloat16)
        y = jnp.dot(xt, g_buf[...], preferred_element_type=jnp.float32)
        o_buf[cur_out] = jnp.maximum(y + b_ref[...], 0.0).astype(o_buf.dtype)

        dma_out(cur_out, step)
        return ()

    jax.lax.fori_loop(0, n_steps, body, (), unroll=False)
    for s in range(max(n_steps - out_slots, 0), n_steps):
        wait_out(s % out_slots)


def kernel(x, w1, b):
    n_data, n_feat = x.shape
    nf_w, n_hidden = w1.shape
    assert nf_w == n_feat

    b2d = b.reshape(1, n_feat)

    block = n_data
    for cand in (1024, 512, 256, 128, 64, 32, 16, 8):
        if n_data % cand == 0:
            block = cand
            break
    n_steps = n_data // block
    in_slots = min(5, max(n_steps, 2))
    out_slots = 3

    cost = pl.CostEstimate(
        flops=2 * n_data * n_feat * n_feat + 2 * n_feat * n_feat * n_hidden,
        transcendentals=0,
        bytes_accessed=2 * n_data * n_feat * 4 + n_feat * n_hidden * 4,
    )
    return pl.pallas_call(
        functools.partial(_fused_kernel, block=block, n_steps=n_steps,
                          in_slots=in_slots, out_slots=out_slots),
        out_shape=jax.ShapeDtypeStruct((n_data, n_feat), x.dtype),
        in_specs=[
            pl.BlockSpec(memory_space=pltpu.MemorySpace.HBM),
            pl.BlockSpec(memory_space=pltpu.MemorySpace.VMEM),
            pl.BlockSpec(memory_space=pltpu.MemorySpace.VMEM),
        ],
        out_specs=pl.BlockSpec(memory_space=pltpu.MemorySpace.HBM),
        scratch_shapes=[
            pltpu.VMEM((in_slots, block, n_feat), x.dtype),
            pltpu.VMEM((out_slots, block, n_feat), x.dtype),
            pltpu.VMEM((n_feat, n_feat), jnp.bfloat16),
            pltpu.SemaphoreType.DMA((in_slots,)),
            pltpu.SemaphoreType.DMA((out_slots,)),
        ],
        cost_estimate=cost,
        compiler_params=pltpu.CompilerParams(
            vmem_limit_bytes=58 * 1024 * 1024,
        ),
    )(x, w1, b2d)


# speedup vs baseline: 1.0711x; 1.0022x over previous
import functools

import jax
import jax.numpy as jnp
from jax.experimental import pallas as pl
from jax.experimental.pallas import tpu as pltpu


def _fused_kernel(x_hbm, w_ref, b_ref, o_hbm,
                  x_buf, o_buf, g_buf, in_sem, out_sem,
                  *, block: int, n_steps: int, in_slots: int, out_slots: int):
    def dma_in(slot, step):
        pltpu.make_async_copy(
            x_hbm.at[pl.ds(step * block, block), :],
            x_buf.at[slot], in_sem.at[slot]).start()

    def wait_in(slot):
        pltpu.make_async_copy(
            x_hbm.at[pl.ds(0, block), :],
            x_buf.at[slot], in_sem.at[slot]).wait()

    def dma_out(slot, step):
        pltpu.make_async_copy(
            o_buf.at[slot],
            o_hbm.at[pl.ds(step * block, block), :], out_sem.at[slot]).start()

    def wait_out(slot):
        pltpu.make_async_copy(
            o_buf.at[slot],
            o_hbm.at[pl.ds(0, block), :], out_sem.at[slot]).wait()

    for s in range(min(in_slots - 1, n_steps)):
        dma_in(s, s)
    w = w_ref[...].astype(jnp.bfloat16)
    g_buf[...] = jax.lax.dot_general(
        w, w, (((1,), (1,)), ((), ())),
        preferred_element_type=jnp.float32).astype(jnp.bfloat16)

    def body(step, _):
        cur_in = jax.lax.rem(step, in_slots)
        cur_out = jax.lax.rem(step, out_slots)

        @pl.when(step + in_slots - 1 < n_steps)
        def _():
            dma_in(jax.lax.rem(step + in_slots - 1, in_slots),
                   step + in_slots - 1)

        wait_in(cur_in)

        @pl.when(step >= out_slots)
        def _():
            wait_out(cur_out)

        xt = x_buf[cur_in].astype(jnp.bfloat16)
        y = jnp.dot(xt, g_buf[...], preferred_element_type=jnp.float32)
        o_buf[cur_out] = jnp.maximum(y + b_ref[...], 0.0).astype(o_buf.dtype)

        dma_out(cur_out, step)
        return ()

    jax.lax.fori_loop(0, n_steps, body, (), unroll=False)
    for s in range(max(n_steps - out_slots, 0), n_steps):
        wait_out(s % out_slots)


def kernel(x, w1, b):
    n_data, n_feat = x.shape
    nf_w, n_hidden = w1.shape
    assert nf_w == n_feat

    b2d = b.reshape(1, n_feat)

    block = n_data
    for cand in (1024, 512, 256, 128, 64, 32, 16, 8):
        if n_data % cand == 0:
            block = cand
            break
    n_steps = n_data // block
    in_slots = min(4, max(n_steps, 2))
    out_slots = 3

    cost = pl.CostEstimate(
        flops=2 * n_data * n_feat * n_feat + 2 * n_feat * n_feat * n_hidden,
        transcendentals=0,
        bytes_accessed=2 * n_data * n_feat * 4 + n_feat * n_hidden * 4,
    )
    return pl.pallas_call(
        functools.partial(_fused_kernel, block=block, n_steps=n_steps,
                          in_slots=in_slots, out_slots=out_slots),
        out_shape=jax.ShapeDtypeStruct((n_data, n_feat), x.dtype),
        in_specs=[
            pl.BlockSpec(memory_space=pltpu.MemorySpace.HBM),
            pl.BlockSpec(memory_space=pltpu.MemorySpace.VMEM),
            pl.BlockSpec(memory_space=pltpu.MemorySpace.VMEM),
        ],
        out_specs=pl.BlockSpec(memory_space=pltpu.MemorySpace.HBM),
        scratch_shapes=[
            pltpu.VMEM((in_slots, block, n_feat), x.dtype),
            pltpu.VMEM((out_slots, block, n_feat), x.dtype),
            pltpu.VMEM((n_feat, n_feat), jnp.bfloat16),
            pltpu.SemaphoreType.DMA((in_slots,)),
            pltpu.SemaphoreType.DMA((out_slots,)),
        ],
        cost_estimate=cost,
        compiler_params=pltpu.CompilerParams(
            vmem_limit_bytes=58 * 1024 * 1024,
        ),
    )(x, w1, b2d)


# speedup vs baseline: 1.0722x; 1.0010x over previous
import functools

import jax
import jax.numpy as jnp
from jax.experimental import pallas as pl
from jax.experimental.pallas import tpu as pltpu


def _fused_kernel(x_hbm, w_ref, b_ref, o_hbm,
                  x_buf, o_buf, g_buf, in_sem, out_sem,
                  *, block: int, n_steps: int, in_slots: int, out_slots: int):
    def dma_in(slot, step):
        pltpu.make_async_copy(
            x_hbm.at[pl.ds(step * block, block), :],
            x_buf.at[slot], in_sem.at[slot]).start()

    def wait_in(slot):
        pltpu.make_async_copy(
            x_hbm.at[pl.ds(0, block), :],
            x_buf.at[slot], in_sem.at[slot]).wait()

    def dma_out(slot, step):
        pltpu.make_async_copy(
            o_buf.at[slot],
            o_hbm.at[pl.ds(step * block, block), :], out_sem.at[slot]).start()

    def wait_out(slot):
        pltpu.make_async_copy(
            o_buf.at[slot],
            o_hbm.at[pl.ds(0, block), :], out_sem.at[slot]).wait()

    for s in range(min(in_slots - 1, n_steps)):
        dma_in(s, s)
    w = w_ref[...].astype(jnp.bfloat16)
    g_buf[...] = jax.lax.dot_general(
        w, w, (((1,), (1,)), ((), ())),
        preferred_element_type=jnp.float32).astype(jnp.bfloat16)

    def body(step, _):
        cur_in = jax.lax.rem(step, in_slots)
        cur_out = jax.lax.rem(step, out_slots)

        @pl.when(step + in_slots - 1 < n_steps)
        def _():
            dma_in(jax.lax.rem(step + in_slots - 1, in_slots),
                   step + in_slots - 1)

        wait_in(cur_in)

        @pl.when(step >= out_slots)
        def _():
            wait_out(cur_out)

        xt = x_buf[cur_in].astype(jnp.bfloat16)
        y = jnp.dot(xt, g_buf[...], preferred_element_type=jnp.float32)
        o_buf[cur_out] = jnp.maximum(y + b_ref[...], 0.0).astype(o_buf.dtype)

        dma_out(cur_out, step)
        return ()

    jax.lax.fori_loop(0, n_steps, body, (), unroll=False)
    for s in range(max(n_steps - out_slots, 0), n_steps):
        wait_out(s % out_slots)


def kernel(x, w1, b):
    n_data, n_feat = x.shape
    nf_w, n_hidden = w1.shape
    assert nf_w == n_feat

    b2d = b.reshape(1, n_feat)

    block = n_data
    for cand in (1024, 512, 256, 128, 64, 32, 16, 8):
        if n_data % cand == 0:
            block = cand
            break
    n_steps = n_data // block
    in_slots = min(5, max(n_steps, 2))
    out_slots = 3

    cost = pl.CostEstimate(
        flops=2 * n_data * n_feat * n_feat + 2 * n_feat * n_feat * n_hidden,
        transcendentals=0,
        bytes_accessed=2 * n_data * n_feat * 4 + n_feat * n_hidden * 4,
    )
    return pl.pallas_call(
        functools.partial(_fused_kernel, block=block, n_steps=n_steps,
                          in_slots=in_slots, out_slots=out_slots),
        out_shape=jax.ShapeDtypeStruct((n_data, n_feat), x.dtype),
        in_specs=[
            pl.BlockSpec(memory_space=pltpu.MemorySpace.HBM),
            pl.BlockSpec(memory_space=pltpu.MemorySpace.VMEM),
            pl.BlockSpec(memory_space=pltpu.MemorySpace.VMEM),
        ],
        out_specs=pl.BlockSpec(memory_space=pltpu.MemorySpace.HBM),
        scratch_shapes=[
            pltpu.VMEM((in_slots, block, n_feat), x.dtype),
            pltpu.VMEM((out_slots, block, n_feat), x.dtype),
            pltpu.VMEM((n_feat, n_feat), jnp.bfloat16),
            pltpu.SemaphoreType.DMA((in_slots,)),
            pltpu.SemaphoreType.DMA((out_slots,)),
        ],
        cost_estimate=cost,
        compiler_params=pltpu.CompilerParams(
            vmem_limit_bytes=58 * 1024 * 1024,
        ),
    )(x, w1, b2d)
